# baseline (speedup 1.0000x reference)
import functools

import jax
import jax.numpy as jnp
from jax import lax
from jax.experimental import pallas as pl
from jax.experimental.pallas import tpu as pltpu
from jax.experimental.pallas import tpu_sc as plsc

BATCH = 4096
SEQ = 200
EMBED_DIM = 64
HIDDEN = 256
VOCAB = 1000000

NUM_CORES = 2
NUM_SUBCORES = 16
NUM_WORKERS = NUM_CORES * NUM_SUBCORES
ROWS_PER_W = BATCH // NUM_WORKERS

CHUNK0 = 128
CHUNK1 = SEQ - CHUNK0

VL = 16
VPR = EMBED_DIM // VL


def _gather_sum_body(x_hbm, emb_hbm, out_hbm, raw_v, idx_v, rows_v, acc_v,
                     sem0, sem1):
    wid = lax.axis_index("s") * NUM_CORES + lax.axis_index("c")
    base = wid * ROWS_PER_W

    pltpu.sync_copy(x_hbm.at[pl.ds(base, ROWS_PER_W)], raw_v)

    offs = tuple(range(0, SEQ - VL, VL)) + (SEQ - VL,)
    log_bt = BT.bit_length() - 1

    def remap_row(b, _):
        for o in offs:
            v = raw_v[b, pl.ds(o, VL)]
            p = jnp.bitwise_and(v, PAIR - 1)
            t = jnp.bitwise_or(
                jnp.left_shift(jnp.bitwise_and(p, BT - 1), 1),
                jnp.right_shift(p, log_bt),
            )
            idx_v[b, pl.ds(o, VL)] = jnp.bitwise_or(
                jnp.bitwise_and(v, ~(PAIR - 1)), t
            )
        return 0

    lax.fori_loop(0, ROWS_PER_W, remap_row, 0)

    sems = (sem0, sem1)

    def start_gather(b, buf, sem):
        pltpu.async_copy(
            emb_hbm.at[idx_v.at[b, pl.ds(0, CHUNK0)]],
            rows_v.at[buf, pl.ds(0, CHUNK0)],
            sem,
        )
        pltpu.async_copy(
            emb_hbm.at[idx_v.at[b, pl.ds(CHUNK0, CHUNK1)]],
            rows_v.at[buf, pl.ds(CHUNK0, CHUNK1)],
            sem,
        )

    def wait_gather(b, buf, sem):
        pltpu.make_async_copy(
            emb_hbm.at[idx_v.at[b, pl.ds(0, CHUNK0)]],
            rows_v.at[buf, pl.ds(0, CHUNK0)],
            sem,
        ).wait()
        pltpu.make_async_copy(
            emb_hbm.at[idx_v.at[b, pl.ds(CHUNK0, CHUNK1)]],
            rows_v.at[buf, pl.ds(CHUNK0, CHUNK1)],
            sem,
        ).wait()

    start_gather(0, 0, sem0)

    def row_body(b, _):
        buf = lax.rem(b, 2)

        @pl.when(b + 1 < ROWS_PER_W)
        def _():
            @pl.when(buf == 0)
            def _():
                start_gather(b + 1, 1, sem1)

            @pl.when(buf == 1)
            def _():
                start_gather(b + 1, 0, sem0)

        @pl.when(buf == 0)
        def _():
            wait_gather(b, 0, sem0)

        @pl.when(buf == 1)
        def _():
            wait_gather(b, 1, sem1)

        def sum_body(j, acc):
            a0, a1, a2, a3 = acc
            w0 = plsc.bitcast(rows_v[buf, j, pl.ds(0, 2 * VL)], jnp.int32)
            w1 = plsc.bitcast(rows_v[buf, j, pl.ds(2 * VL, 2 * VL)], jnp.int32)
            a0 = a0 + plsc.bitcast(jnp.left_shift(w0, 16), jnp.float32)
            a1 = a1 + plsc.bitcast(jnp.bitwise_and(w0, -65536), jnp.float32)
            a2 = a2 + plsc.bitcast(jnp.left_shift(w1, 16), jnp.float32)
            a3 = a3 + plsc.bitcast(jnp.bitwise_and(w1, -65536), jnp.float32)
            return (a0, a1, a2, a3)

        zero = jnp.zeros((VL,), jnp.float32)
        acc = lax.fori_loop(0, SEQ, sum_body, (zero,) * VPR, unroll=8)
        for c in range(VPR):
            acc_v[b, pl.ds(c * VL, VL)] = acc[c]
        return 0

    lax.fori_loop(0, ROWS_PER_W, row_body, 0)

    pltpu.sync_copy(acc_v, out_hbm.at[pl.ds(base, ROWS_PER_W)])


@jax.jit
def _gather_sum(x, emb):
    mesh = plsc.VectorSubcoreMesh(
        core_axis_name="c", subcore_axis_name="s",
        num_cores=NUM_CORES, num_subcores=NUM_SUBCORES,
    )
    return pl.kernel(
        _gather_sum_body,
        out_type=jax.ShapeDtypeStruct((BATCH, EMBED_DIM), jnp.float32),
        mesh=mesh,
        compiler_params=pltpu.CompilerParams(use_tc_tiling_on_sc=False,
                                             needs_layout_passes=False),
        scratch_types=[
            pltpu.VMEM((ROWS_PER_W, SEQ), jnp.int32),
            pltpu.VMEM((ROWS_PER_W, SEQ), jnp.int32),
            pltpu.VMEM((2, SEQ, EMBED_DIM), jnp.bfloat16),
            pltpu.VMEM((ROWS_PER_W, EMBED_DIM), jnp.float32),
            pltpu.SemaphoreType.DMA,
            pltpu.SemaphoreType.DMA,
        ],
    )(x, emb)


BT = 4096
PAIR = 2 * BT
N_PAIRS = (VOCAB + PAIR - 1) // PAIR
ROWS_OUT = N_PAIRS * PAIR


def _transpose_body(x0_ref, x1_ref, out_ref):
    y = jnp.concatenate([x0_ref[...], x1_ref[...]], axis=0)
    out_ref[...] = y.T.astype(jnp.bfloat16).reshape(out_ref.shape)


@jax.jit
def _linearize_table(embT):
    packed = pl.pallas_call(
        _transpose_body,
        grid=(N_PAIRS,),
        in_specs=[
            pl.BlockSpec((EMBED_DIM, BT), lambda i: (0, 2 * i)),
            pl.BlockSpec((EMBED_DIM, BT),
                         lambda i: (0, jnp.minimum(2 * i + 1, 2 * N_PAIRS - 2))),
        ],
        out_specs=pl.BlockSpec((BT * 2 * EMBED_DIM,), lambda i: (i,)),
        out_shape=jax.ShapeDtypeStruct((N_PAIRS * BT * 2 * EMBED_DIM,),
                                       jnp.bfloat16),
    )(embT, embT)
    return packed.reshape(ROWS_OUT, EMBED_DIM)


def _mlp_body(summed_ref, x_ref, emb0_ref, w1_ref, b1_ref, w2_ref, b2_ref,
              out_ref):
    xb = x_ref[...]
    nnz = jnp.sum((xb != 0).astype(jnp.float32), axis=1, keepdims=True)
    n_zeros = jnp.float32(SEQ) - nnz
    avg = (summed_ref[...] - n_zeros * emb0_ref[...]) / nnz
    h = jnp.dot(avg, w1_ref[...], preferred_element_type=jnp.float32)
    h = jnp.maximum(h + b1_ref[...], 0.0)
    logits = jnp.dot(h, w2_ref[...], preferred_element_type=jnp.float32)
    logits = logits + b2_ref[...]
    m = jnp.max(logits, axis=1, keepdims=True)
    lse = m + jnp.log(jnp.sum(jnp.exp(logits - m), axis=1, keepdims=True))
    out_ref[...] = logits - lse


@jax.jit
def _mlp(summed, x, emb0, w1t, b1, w2t, b2):
    blk = 512
    grid = BATCH // blk
    return pl.pallas_call(
        _mlp_body,
        grid=(grid,),
        in_specs=[
            pl.BlockSpec((blk, EMBED_DIM), lambda i: (i, 0)),
            pl.BlockSpec((blk, SEQ), lambda i: (i, 0)),
            pl.BlockSpec((1, EMBED_DIM), lambda i: (0, 0)),
            pl.BlockSpec((EMBED_DIM, HIDDEN), lambda i: (0, 0)),
            pl.BlockSpec((1, HIDDEN), lambda i: (0, 0)),
            pl.BlockSpec((HIDDEN, 2), lambda i: (0, 0)),
            pl.BlockSpec((1, 2), lambda i: (0, 0)),
        ],
        out_specs=pl.BlockSpec((blk, 2), lambda i: (i, 0)),
        out_shape=jax.ShapeDtypeStruct((BATCH, 2), jnp.float32),
    )(summed, x, emb0, w1t, b1, w2t, b2)


_COL_PERM = (tuple(range(0, 32, 2)) + tuple(range(1, 32, 2))
             + tuple(range(32, 64, 2)) + tuple(range(33, 64, 2)))


def kernel(x, emb, W1, b1, W2, b2):
    x = x.astype(jnp.int32)
    embL = _linearize_table(emb.T)
    summed = _gather_sum(x, embL)
    perm = jnp.asarray(_COL_PERM, dtype=jnp.int32)
    emb0 = lax.slice(embL, (0, 0), (1, EMBED_DIM)).astype(jnp.float32)[:, perm]
    w1t = W1.T[perm, :]
    return _mlp(summed, x, emb0, w1t, b1[None, :], W2.T, b2[None, :])


# speedup vs baseline: 1.9705x; 1.9705x over previous
import functools

import jax
import jax.numpy as jnp
from jax import lax
from jax.experimental import pallas as pl
from jax.experimental.pallas import tpu as pltpu
from jax.experimental.pallas import tpu_sc as plsc

BATCH = 4096
SEQ = 200
EMBED_DIM = 64
HIDDEN = 256
VOCAB = 1000000

NUM_CORES = 2
NUM_SUBCORES = 16
NUM_WORKERS = NUM_CORES * NUM_SUBCORES
ROWS_PER_W = BATCH // NUM_WORKERS

CHUNK0 = 128
CHUNK1 = SEQ - CHUNK0

VL = 16
VPR = EMBED_DIM // VL


def _gather_sum_body(x_hbm, emb_hbm, out_hbm, raw_v, idx_v, rows_v, acc_v,
                     sem0, sem1):
    wid = lax.axis_index("s") * NUM_CORES + lax.axis_index("c")
    base = wid * ROWS_PER_W

    pltpu.sync_copy(x_hbm.at[pl.ds(base, ROWS_PER_W)], raw_v)

    offs = tuple(range(0, SEQ - VL, VL)) + (SEQ - VL,)
    log_bt = BT.bit_length() - 1

    def remap_row(b, _):
        for o in offs:
            v = raw_v[b, pl.ds(o, VL)]
            p = jnp.bitwise_and(v, PAIR - 1)
            h = jnp.right_shift(p, log_bt)
            k = jnp.bitwise_and(p, BT - 1)
            q = jnp.bitwise_or(
                jnp.left_shift(jnp.bitwise_and(k, BT // 2 - 1), 2),
                jnp.bitwise_or(
                    jnp.left_shift(jnp.right_shift(k, log_bt - 1), 1), h),
            )
            idx_v[b, pl.ds(o, VL)] = jnp.bitwise_or(
                jnp.bitwise_and(v, ~(PAIR - 1)), q
            )
        return 0

    lax.fori_loop(0, ROWS_PER_W, remap_row, 0)

    sems = (sem0, sem1)

    def start_gather(b, buf, sem):
        pltpu.async_copy(
            emb_hbm.at[idx_v.at[b, pl.ds(0, CHUNK0)]],
            rows_v.at[buf, pl.ds(0, CHUNK0)],
            sem,
        )
        pltpu.async_copy(
            emb_hbm.at[idx_v.at[b, pl.ds(CHUNK0, CHUNK1)]],
            rows_v.at[buf, pl.ds(CHUNK0, CHUNK1)],
            sem,
        )

    def wait_gather(b, buf, sem):
        pltpu.make_async_copy(
            emb_hbm.at[idx_v.at[b, pl.ds(0, CHUNK0)]],
            rows_v.at[buf, pl.ds(0, CHUNK0)],
            sem,
        ).wait()
        pltpu.make_async_copy(
            emb_hbm.at[idx_v.at[b, pl.ds(CHUNK0, CHUNK1)]],
            rows_v.at[buf, pl.ds(CHUNK0, CHUNK1)],
            sem,
        ).wait()

    start_gather(0, 0, sem0)

    def row_body(b, _):
        buf = lax.rem(b, 2)

        @pl.when(b + 1 < ROWS_PER_W)
        def _():
            @pl.when(buf == 0)
            def _():
                start_gather(b + 1, 1, sem1)

            @pl.when(buf == 1)
            def _():
                start_gather(b + 1, 0, sem0)

        @pl.when(buf == 0)
        def _():
            wait_gather(b, 0, sem0)

        @pl.when(buf == 1)
        def _():
            wait_gather(b, 1, sem1)

        def sum_body(j, acc):
            a0, a1, a2, a3 = acc
            w0 = rows_v[buf, j, pl.ds(0, VL)]
            w1 = rows_v[buf, j, pl.ds(VL, VL)]
            a0 = a0 + plsc.bitcast(jnp.left_shift(w0, 16), jnp.float32)
            a1 = a1 + plsc.bitcast(jnp.bitwise_and(w0, -65536), jnp.float32)
            a2 = a2 + plsc.bitcast(jnp.left_shift(w1, 16), jnp.float32)
            a3 = a3 + plsc.bitcast(jnp.bitwise_and(w1, -65536), jnp.float32)
            return (a0, a1, a2, a3)

        zero = jnp.zeros((VL,), jnp.float32)
        acc = lax.fori_loop(0, SEQ, sum_body, (zero,) * VPR, unroll=8)
        for c in range(VPR):
            acc_v[b, pl.ds(c * VL, VL)] = acc[c]
        return 0

    lax.fori_loop(0, ROWS_PER_W, row_body, 0)

    pltpu.sync_copy(acc_v, out_hbm.at[pl.ds(base, ROWS_PER_W)])


@jax.jit
def _gather_sum(x, emb):
    mesh = plsc.VectorSubcoreMesh(
        core_axis_name="c", subcore_axis_name="s",
        num_cores=NUM_CORES, num_subcores=NUM_SUBCORES,
    )
    return pl.kernel(
        _gather_sum_body,
        out_type=jax.ShapeDtypeStruct((BATCH, EMBED_DIM), jnp.float32),
        mesh=mesh,
        compiler_params=pltpu.CompilerParams(use_tc_tiling_on_sc=False,
                                             needs_layout_passes=False),
        scratch_types=[
            pltpu.VMEM((ROWS_PER_W, SEQ), jnp.int32),
            pltpu.VMEM((ROWS_PER_W, SEQ), jnp.int32),
            pltpu.VMEM((2, SEQ, EMBED_DIM // 2), jnp.int32),
            pltpu.VMEM((ROWS_PER_W, EMBED_DIM), jnp.float32),
            pltpu.SemaphoreType.DMA,
            pltpu.SemaphoreType.DMA,
        ],
    )(x, emb)


BT = 4096
PAIR = 2 * BT
N_PAIRS = (VOCAB + PAIR - 1) // PAIR
ROWS_OUT = N_PAIRS * PAIR


import numpy as _np

_PERM_MAT = _np.zeros((128, 128), _np.float32)
for _half in (0, 64):
    for _l in range(32):
        _PERM_MAT[_half + 2 * _l, _half + _l] = 1.0
        _PERM_MAT[_half + 2 * _l + 1, _half + 32 + _l] = 1.0


def _transpose_body(x0_ref, x1_ref, pmat_ref, out_ref):
    y = jnp.concatenate([x0_ref[...], x1_ref[...]], axis=0)
    yb = y.astype(jnp.bfloat16)
    z = jnp.dot(yb.T, pmat_ref[...], preferred_element_type=jnp.float32)
    u = jax.lax.bitcast_convert_type(z, jnp.uint32)
    r = jnp.right_shift(u, 16)
    lo = jax.lax.slice_in_dim(r, 0, 32, axis=1)
    hi = jax.lax.slice_in_dim(r, 32, 64, axis=1)
    lo2 = jax.lax.slice_in_dim(r, 64, 96, axis=1)
    hi2 = jax.lax.slice_in_dim(r, 96, 128, axis=1)
    w = jnp.concatenate(
        [jnp.bitwise_or(lo, jnp.left_shift(hi, 16)),
         jnp.bitwise_or(lo2, jnp.left_shift(hi2, 16))], axis=1)
    out_ref[...] = jax.lax.bitcast_convert_type(
        jnp.concatenate([w[: BT // 2, :], w[BT // 2:, :]], axis=1), jnp.int32)


@jax.jit
def _linearize_table(embT):
    packed = pl.pallas_call(
        _transpose_body,
        grid=(N_PAIRS,),
        in_specs=[
            pl.BlockSpec((EMBED_DIM, BT), lambda i: (0, 2 * i)),
            pl.BlockSpec((EMBED_DIM, BT),
                         lambda i: (0, jnp.minimum(2 * i + 1, 2 * N_PAIRS - 2))),
            pl.BlockSpec((128, 128), lambda i: (0, 0)),
        ],
        out_specs=pl.BlockSpec((BT // 2, 128), lambda i: (i, 0)),
        out_shape=jax.ShapeDtypeStruct((N_PAIRS * BT // 2, 128), jnp.int32),
        compiler_params=pltpu.CompilerParams(
            fuse_transposed_lhs_in_matmul=True),
    )(embT, embT, jnp.asarray(_PERM_MAT, dtype=jnp.bfloat16))
    return packed.reshape(N_PAIRS * BT * 64).reshape(ROWS_OUT, 32)


def _mlp_body(summed_ref, x_ref, emb0_ref, w1_ref, b1_ref, w2_ref, b2_ref,
              out_ref):
    xb = x_ref[...]
    nnz = jnp.sum((xb != 0).astype(jnp.float32), axis=1, keepdims=True)
    n_zeros = jnp.float32(SEQ) - nnz
    avg = (summed_ref[...] - n_zeros * emb0_ref[...]) / nnz
    h = jnp.dot(avg, w1_ref[...], preferred_element_type=jnp.float32)
    h = jnp.maximum(h + b1_ref[...], 0.0)
    logits = jnp.dot(h, w2_ref[...], preferred_element_type=jnp.float32)
    logits = logits + b2_ref[...]
    m = jnp.max(logits, axis=1, keepdims=True)
    lse = m + jnp.log(jnp.sum(jnp.exp(logits - m), axis=1, keepdims=True))
    out_ref[...] = logits - lse


@jax.jit
def _mlp(summed, x, emb0, w1t, b1, w2t, b2):
    blk = 512
    grid = BATCH // blk
    return pl.pallas_call(
        _mlp_body,
        grid=(grid,),
        in_specs=[
            pl.BlockSpec((blk, EMBED_DIM), lambda i: (i, 0)),
            pl.BlockSpec((blk, SEQ), lambda i: (i, 0)),
            pl.BlockSpec((1, EMBED_DIM), lambda i: (0, 0)),
            pl.BlockSpec((EMBED_DIM, HIDDEN), lambda i: (0, 0)),
            pl.BlockSpec((1, HIDDEN), lambda i: (0, 0)),
            pl.BlockSpec((HIDDEN, 2), lambda i: (0, 0)),
            pl.BlockSpec((1, 2), lambda i: (0, 0)),
        ],
        out_specs=pl.BlockSpec((blk, 2), lambda i: (i, 0)),
        out_shape=jax.ShapeDtypeStruct((BATCH, 2), jnp.float32),
    )(summed, x, emb0, w1t, b1, w2t, b2)


_COL_PERM = (tuple(range(0, 32, 2)) + tuple(range(1, 32, 2))
             + tuple(range(32, 64, 2)) + tuple(range(33, 64, 2)))


def kernel(x, emb, W1, b1, W2, b2):
    x = x.astype(jnp.int32)
    embL = _linearize_table(emb.T)
    summed = _gather_sum(x, embL)
    perm = jnp.asarray(_COL_PERM, dtype=jnp.int32)
    w = lax.slice(embL, (0, 0), (1, EMBED_DIM // 2))
    ev = lax.bitcast_convert_type(jnp.left_shift(w, 16), jnp.float32)
    od = lax.bitcast_convert_type(jnp.bitwise_and(w, -65536), jnp.float32)
    emb0 = jnp.concatenate(
        [ev[:, 0:16], od[:, 0:16], ev[:, 16:32], od[:, 16:32]], axis=1)
    w1t = W1.T[perm, :]
    return _mlp(summed, x, emb0, w1t, b1[None, :], W2.T, b2[None, :])


# speedup vs baseline: 2.1997x; 1.1163x over previous
import functools

import jax
import jax.numpy as jnp
from jax import lax
from jax.experimental import pallas as pl
from jax.experimental.pallas import tpu as pltpu
from jax.experimental.pallas import tpu_sc as plsc

BATCH = 4096
SEQ = 200
EMBED_DIM = 64
HIDDEN = 256
VOCAB = 1000000

NUM_CORES = 2
NUM_SUBCORES = 16
NUM_WORKERS = NUM_CORES * NUM_SUBCORES
ROWS_PER_W = BATCH // NUM_WORKERS

CHUNK0 = 128
CHUNK1 = SEQ - CHUNK0

VL = 16
VPR = EMBED_DIM // VL


def _gather_sum_body(x_hbm, emb_hbm, out_hbm, raw_v, idx_v, rows_v, acc_v,
                     sem0, sem1):
    wid = lax.axis_index("s") * NUM_CORES + lax.axis_index("c")
    base = wid * ROWS_PER_W

    pltpu.sync_copy(x_hbm.at[pl.ds(base, ROWS_PER_W)], raw_v)

    offs = tuple(range(0, SEQ - VL, VL)) + (SEQ - VL,)
    log_bt = BT.bit_length() - 1

    def remap_row(b, _):
        for o in offs:
            v = raw_v[b, pl.ds(o, VL)]
            p = jnp.bitwise_and(v, PAIR - 1)
            h = jnp.right_shift(p, log_bt)
            k = jnp.bitwise_and(p, BT - 1)
            q = jnp.bitwise_or(
                jnp.left_shift(jnp.bitwise_and(k, BT // 2 - 1), 2),
                jnp.bitwise_or(
                    jnp.left_shift(jnp.right_shift(k, log_bt - 1), 1), h),
            )
            idx_v[b, pl.ds(o, VL)] = jnp.bitwise_or(
                jnp.bitwise_and(v, ~(PAIR - 1)), q
            )
        return 0

    lax.fori_loop(0, ROWS_PER_W, remap_row, 0)

    sems = (sem0, sem1)

    def start_gather(b, buf, sem):
        pltpu.async_copy(
            emb_hbm.at[idx_v.at[b, pl.ds(0, CHUNK0)]],
            rows_v.at[buf, pl.ds(0, CHUNK0)],
            sem,
        )
        pltpu.async_copy(
            emb_hbm.at[idx_v.at[b, pl.ds(CHUNK0, CHUNK1)]],
            rows_v.at[buf, pl.ds(CHUNK0, CHUNK1)],
            sem,
        )

    def wait_gather(b, buf, sem):
        pltpu.make_async_copy(
            emb_hbm.at[idx_v.at[b, pl.ds(0, CHUNK0)]],
            rows_v.at[buf, pl.ds(0, CHUNK0)],
            sem,
        ).wait()
        pltpu.make_async_copy(
            emb_hbm.at[idx_v.at[b, pl.ds(CHUNK0, CHUNK1)]],
            rows_v.at[buf, pl.ds(CHUNK0, CHUNK1)],
            sem,
        ).wait()

    start_gather(0, 0, sem0)

    def row_body(b, _):
        buf = lax.rem(b, 2)

        @pl.when(b + 1 < ROWS_PER_W)
        def _():
            @pl.when(buf == 0)
            def _():
                start_gather(b + 1, 1, sem1)

            @pl.when(buf == 1)
            def _():
                start_gather(b + 1, 0, sem0)

        @pl.when(buf == 0)
        def _():
            wait_gather(b, 0, sem0)

        @pl.when(buf == 1)
        def _():
            wait_gather(b, 1, sem1)

        def sum_body(j, acc):
            a0, a1, a2, a3 = acc
            w0 = rows_v[buf, j, pl.ds(0, VL)]
            w1 = rows_v[buf, j, pl.ds(VL, VL)]
            a0 = a0 + plsc.bitcast(jnp.left_shift(w0, 16), jnp.float32)
            a1 = a1 + plsc.bitcast(jnp.bitwise_and(w0, -65536), jnp.float32)
            a2 = a2 + plsc.bitcast(jnp.left_shift(w1, 16), jnp.float32)
            a3 = a3 + plsc.bitcast(jnp.bitwise_and(w1, -65536), jnp.float32)
            return (a0, a1, a2, a3)

        zero = jnp.zeros((VL,), jnp.float32)
        acc = lax.fori_loop(0, SEQ, sum_body, (zero,) * VPR, unroll=8)
        for c in range(VPR):
            acc_v[b, pl.ds(c * VL, VL)] = acc[c]
        return 0

    lax.fori_loop(0, ROWS_PER_W, row_body, 0)

    pltpu.sync_copy(acc_v, out_hbm.at[pl.ds(base, ROWS_PER_W)])


@jax.jit
def _gather_sum(x, emb):
    mesh = plsc.VectorSubcoreMesh(
        core_axis_name="c", subcore_axis_name="s",
        num_cores=NUM_CORES, num_subcores=NUM_SUBCORES,
    )
    return pl.kernel(
        _gather_sum_body,
        out_type=jax.ShapeDtypeStruct((BATCH, EMBED_DIM), jnp.float32),
        mesh=mesh,
        compiler_params=pltpu.CompilerParams(use_tc_tiling_on_sc=False,
                                             needs_layout_passes=False),
        scratch_types=[
            pltpu.VMEM((ROWS_PER_W, SEQ), jnp.int32),
            pltpu.VMEM((ROWS_PER_W, SEQ), jnp.int32),
            pltpu.VMEM((2, SEQ, EMBED_DIM // 2), jnp.int32),
            pltpu.VMEM((ROWS_PER_W, EMBED_DIM), jnp.float32),
            pltpu.SemaphoreType.DMA,
            pltpu.SemaphoreType.DMA,
        ],
    )(x, emb)


BT = 8192
PAIR = 2 * BT
N_PAIRS = (VOCAB + PAIR - 1) // PAIR
ROWS_OUT = N_PAIRS * PAIR


import numpy as _np

_PERM_MAT = _np.zeros((128, 128), _np.float32)
for _half in (0, 64):
    for _l in range(32):
        _PERM_MAT[_half + 2 * _l, _half + _l] = 1.0
        _PERM_MAT[_half + 2 * _l + 1, _half + 32 + _l] = 1.0


def _transpose_body(x0_ref, x1_ref, pmat_ref, out_ref):
    y = jnp.concatenate([x0_ref[...], x1_ref[...]], axis=0)
    yb = y.astype(jnp.bfloat16)
    z = jnp.dot(yb.T, pmat_ref[...], preferred_element_type=jnp.float32)
    u = jax.lax.bitcast_convert_type(z, jnp.uint32)
    r = jnp.right_shift(u, 16)
    lo = jax.lax.slice_in_dim(r, 0, 32, axis=1)
    hi = jax.lax.slice_in_dim(r, 32, 64, axis=1)
    lo2 = jax.lax.slice_in_dim(r, 64, 96, axis=1)
    hi2 = jax.lax.slice_in_dim(r, 96, 128, axis=1)
    w = jnp.concatenate(
        [jnp.bitwise_or(lo, jnp.left_shift(hi, 16)),
         jnp.bitwise_or(lo2, jnp.left_shift(hi2, 16))], axis=1)
    out_ref[...] = jax.lax.bitcast_convert_type(
        jnp.concatenate([w[: BT // 2, :], w[BT // 2:, :]], axis=1), jnp.int32)


@jax.jit
def _linearize_table(embT):
    packed = pl.pallas_call(
        _transpose_body,
        grid=(N_PAIRS,),
        in_specs=[
            pl.BlockSpec((EMBED_DIM, BT), lambda i: (0, 2 * i)),
            pl.BlockSpec((EMBED_DIM, BT),
                         lambda i: (0, jnp.minimum(2 * i + 1, 2 * N_PAIRS - 2))),
            pl.BlockSpec((128, 128), lambda i: (0, 0)),
        ],
        out_specs=pl.BlockSpec((BT // 2, 128), lambda i: (i, 0)),
        out_shape=jax.ShapeDtypeStruct((N_PAIRS * BT // 2, 128), jnp.int32),
        compiler_params=pltpu.CompilerParams(
            fuse_transposed_lhs_in_matmul=True),
    )(embT, embT, jnp.asarray(_PERM_MAT, dtype=jnp.bfloat16))
    return packed.reshape(N_PAIRS * BT * 64).reshape(ROWS_OUT, 32)


def _mlp_body(summed_ref, x_ref, emb0_ref, w1_ref, b1_ref, w2_ref, b2_ref,
              out_ref):
    xb = x_ref[...]
    nnz = jnp.sum((xb != 0).astype(jnp.float32), axis=1, keepdims=True)
    n_zeros = jnp.float32(SEQ) - nnz
    avg = (summed_ref[...] - n_zeros * emb0_ref[...]) / nnz
    h = jnp.dot(avg, w1_ref[...], preferred_element_type=jnp.float32)
    h = jnp.maximum(h + b1_ref[...], 0.0)
    logits = jnp.dot(h, w2_ref[...], preferred_element_type=jnp.float32)
    logits = logits + b2_ref[...]
    m = jnp.max(logits, axis=1, keepdims=True)
    lse = m + jnp.log(jnp.sum(jnp.exp(logits - m), axis=1, keepdims=True))
    out_ref[...] = logits - lse


@jax.jit
def _mlp(summed, x, emb0, w1t, b1, w2t, b2):
    blk = 512
    grid = BATCH // blk
    return pl.pallas_call(
        _mlp_body,
        grid=(grid,),
        in_specs=[
            pl.BlockSpec((blk, EMBED_DIM), lambda i: (i, 0)),
            pl.BlockSpec((blk, SEQ), lambda i: (i, 0)),
            pl.BlockSpec((1, EMBED_DIM), lambda i: (0, 0)),
            pl.BlockSpec((EMBED_DIM, HIDDEN), lambda i: (0, 0)),
            pl.BlockSpec((1, HIDDEN), lambda i: (0, 0)),
            pl.BlockSpec((HIDDEN, 2), lambda i: (0, 0)),
            pl.BlockSpec((1, 2), lambda i: (0, 0)),
        ],
        out_specs=pl.BlockSpec((blk, 2), lambda i: (i, 0)),
        out_shape=jax.ShapeDtypeStruct((BATCH, 2), jnp.float32),
    )(summed, x, emb0, w1t, b1, w2t, b2)


_COL_PERM = (tuple(range(0, 32, 2)) + tuple(range(1, 32, 2))
             + tuple(range(32, 64, 2)) + tuple(range(33, 64, 2)))


def kernel(x, emb, W1, b1, W2, b2):
    x = x.astype(jnp.int32)
    embL = _linearize_table(emb.T)
    summed = _gather_sum(x, embL)
    perm = jnp.asarray(_COL_PERM, dtype=jnp.int32)
    w = lax.slice(embL, (0, 0), (1, EMBED_DIM // 2))
    ev = lax.bitcast_convert_type(jnp.left_shift(w, 16), jnp.float32)
    od = lax.bitcast_convert_type(jnp.bitwise_and(w, -65536), jnp.float32)
    emb0 = jnp.concatenate(
        [ev[:, 0:16], od[:, 0:16], ev[:, 16:32], od[:, 16:32]], axis=1)
    w1t = W1.T[perm, :]
    return _mlp(summed, x, emb0, w1t, b1[None, :], W2.T, b2[None, :])
